# Initial kernel scaffold; baseline (speedup 1.0000x reference)
import functools

import jax
import jax.numpy as jnp
from jax import lax
from jax.experimental import pallas as pl
from jax.experimental.pallas import tpu as pltpu
from jax.experimental.pallas import tpu_sc as plsc

NUM_EMBEDDINGS = 100000
EMBED_DIM = 128
BATCH = 16
SEQ = 2048

_NC = 2
_NS = 16
_NW = _NC * _NS

_B = BATCH * SEQ
_BPW = _B // _NW
_CHUNK = 256
_NCHUNK = _BPW // _CHUNK


def _make_gather():
    mesh = plsc.VectorSubcoreMesh(core_axis_name="c", subcore_axis_name="s")

    @functools.partial(
        pl.kernel,
        mesh=mesh,
        out_type=jax.ShapeDtypeStruct((_B, EMBED_DIM), jnp.float32),
        scratch_types=[
            pltpu.VMEM((_BPW,), jnp.int32),
            pltpu.VMEM((2, _CHUNK, EMBED_DIM), jnp.float32),
            pltpu.SemaphoreType.DMA,
            pltpu.SemaphoreType.DMA,
        ],
    )
    def gather_kernel(idx_hbm, table_hbm, out_hbm, idx_v, rows_v, gsem, osem):
        wid = lax.axis_index("s") * _NC + lax.axis_index("c")
        base = wid * _BPW
        pltpu.sync_copy(idx_hbm.at[pl.ds(base, _BPW)], idx_v)

        gathers = [None, None]
        stores = [None, None]
        for i in range(_NCHUNK):
            b = i % 2
            if stores[b] is not None:
                stores[b].wait()
            gathers[b] = pltpu.async_copy(
                table_hbm.at[idx_v.at[pl.ds(i * _CHUNK, _CHUNK)]],
                rows_v.at[b],
                gsem,
            )
            if i > 0:
                pb = (i - 1) % 2
                gathers[pb].wait()
                stores[pb] = pltpu.async_copy(
                    rows_v.at[pb],
                    out_hbm.at[pl.ds(base + (i - 1) * _CHUNK, _CHUNK)],
                    osem,
                )
        last = (_NCHUNK - 1) % 2
        gathers[last].wait()
        stores[last] = pltpu.async_copy(
            rows_v.at[last],
            out_hbm.at[pl.ds(base + (_NCHUNK - 1) * _CHUNK, _CHUNK)],
            osem,
        )
        for s in stores:
            if s is not None:
                s.wait()

    return gather_kernel


_gather = _make_gather()


@jax.jit
def kernel(input, table):
    idx = jnp.reshape(input, (_B,)).astype(jnp.int32)
    out = _gather(idx, table)
    return jnp.reshape(out, (BATCH, SEQ, EMBED_DIM))


# speedup vs baseline: 1.3994x; 1.3994x over previous
import functools

import jax
import jax.numpy as jnp
from jax import lax
from jax.experimental import pallas as pl
from jax.experimental.pallas import tpu as pltpu
from jax.experimental.pallas import tpu_sc as plsc

NUM_EMBEDDINGS = 100000
EMBED_DIM = 128
BATCH = 16
SEQ = 2048

_NC = 2
_NS = 16
_NW = _NC * _NS

_B = BATCH * SEQ
_BPW = _B // _NW
_CHUNK = 128
_NCHUNK = _BPW // _CHUNK


def _make_gather():
    mesh = plsc.VectorSubcoreMesh(core_axis_name="c", subcore_axis_name="s")

    @functools.partial(
        pl.kernel,
        mesh=mesh,
        out_type=jax.ShapeDtypeStruct((_B, EMBED_DIM), jnp.float32),
        scratch_types=[
            pltpu.VMEM((_BPW,), jnp.int32),
            pltpu.VMEM((2, _CHUNK, EMBED_DIM), jnp.float32),
            pltpu.SemaphoreType.DMA,
            pltpu.SemaphoreType.DMA,
        ],
    )
    def gather_kernel(idx_hbm, table_hbm, out_hbm, idx_v, rows_v, gsem, osem):
        wid = lax.axis_index("s") * _NC + lax.axis_index("c")
        base = wid * _BPW
        pltpu.sync_copy(idx_hbm.at[pl.ds(base, _BPW)], idx_v)

        gathers = [None, None]
        stores = [None, None]
        for i in range(_NCHUNK):
            b = i % 2
            if stores[b] is not None:
                stores[b].wait()
            gathers[b] = pltpu.async_copy(
                table_hbm.at[idx_v.at[pl.ds(i * _CHUNK, _CHUNK)]],
                rows_v.at[b],
                gsem,
            )
            if i > 0:
                pb = (i - 1) % 2
                gathers[pb].wait()
                stores[pb] = pltpu.async_copy(
                    rows_v.at[pb],
                    out_hbm.at[pl.ds(base + (i - 1) * _CHUNK, _CHUNK)],
                    osem,
                )
        last = (_NCHUNK - 1) % 2
        gathers[last].wait()
        stores[last] = pltpu.async_copy(
            rows_v.at[last],
            out_hbm.at[pl.ds(base + (_NCHUNK - 1) * _CHUNK, _CHUNK)],
            osem,
        )
        for s in stores:
            if s is not None:
                s.wait()

    return gather_kernel


_gather = _make_gather()


@jax.jit
def kernel(input, table):
    idx = jnp.reshape(input, (_B,)).astype(jnp.int32)
    out = _gather(idx, table)
    return jnp.reshape(out, (BATCH, SEQ, EMBED_DIM))


# speedup vs baseline: 1.4334x; 1.0243x over previous
import functools

import jax
import jax.numpy as jnp
from jax import lax
from jax.experimental import pallas as pl
from jax.experimental.pallas import tpu as pltpu
from jax.experimental.pallas import tpu_sc as plsc

NUM_EMBEDDINGS = 100000
EMBED_DIM = 128
BATCH = 16
SEQ = 2048

_NC = 2
_NS = 16
_NW = _NC * _NS

_B = BATCH * SEQ
_BPW = _B // _NW
_CHUNK = 256
_NCHUNK = _BPW // _CHUNK


def _make_gather():
    mesh = plsc.VectorSubcoreMesh(core_axis_name="c", subcore_axis_name="s")

    @functools.partial(
        pl.kernel,
        mesh=mesh,
        out_type=jax.ShapeDtypeStruct((_B, EMBED_DIM), jnp.float32),
        scratch_types=[
            pltpu.VMEM((_BPW,), jnp.int32),
            pltpu.VMEM((2, _CHUNK, EMBED_DIM), jnp.float32),
            pltpu.SemaphoreType.DMA,
            pltpu.SemaphoreType.DMA,
        ],
    )
    def gather_kernel(idx_hbm, table_hbm, out_hbm, idx_v, rows_v, gsem, osem):
        wid = lax.axis_index("s") * _NC + lax.axis_index("c")
        base = wid * _BPW
        pltpu.sync_copy(idx_hbm.at[pl.ds(base, _BPW)], idx_v)

        gathers = [None, None]
        stores = [None, None]
        for i in range(_NCHUNK):
            b = i % 2
            if stores[b] is not None:
                stores[b].wait()
            gathers[b] = pltpu.async_copy(
                table_hbm.at[idx_v.at[pl.ds(i * _CHUNK, _CHUNK)]],
                rows_v.at[b],
                gsem,
            )
            if i > 0:
                pb = (i - 1) % 2
                gathers[pb].wait()
                stores[pb] = pltpu.async_copy(
                    rows_v.at[pb],
                    out_hbm.at[pl.ds(base + (i - 1) * _CHUNK, _CHUNK)],
                    osem,
                )
        last = (_NCHUNK - 1) % 2
        gathers[last].wait()
        stores[last] = pltpu.async_copy(
            rows_v.at[last],
            out_hbm.at[pl.ds(base + (_NCHUNK - 1) * _CHUNK, _CHUNK)],
            osem,
        )
        for s in stores:
            if s is not None:
                s.wait()

    return gather_kernel


_gather = _make_gather()


@jax.jit
def kernel(input, table):
    idx = jnp.reshape(input, (_B,)).astype(jnp.int32)
    out = _gather(idx, table)
    return jnp.reshape(out, (BATCH, SEQ, EMBED_DIM))


# speedup vs baseline: 1.4374x; 1.0028x over previous
import functools

import jax
import jax.numpy as jnp
from jax import lax
from jax.experimental import pallas as pl
from jax.experimental.pallas import tpu as pltpu
from jax.experimental.pallas import tpu_sc as plsc

NUM_EMBEDDINGS = 100000
EMBED_DIM = 128
BATCH = 16
SEQ = 2048

_NC = 2
_NS = 16
_NW = _NC * _NS

_B = BATCH * SEQ
_BPW = _B // _NW
_CHUNK = 256
_NCHUNK = _BPW // _CHUNK
_NBUF = 3


def _make_gather():
    mesh = plsc.VectorSubcoreMesh(core_axis_name="c", subcore_axis_name="s")

    @functools.partial(
        pl.kernel,
        mesh=mesh,
        out_type=jax.ShapeDtypeStruct((_B, EMBED_DIM), jnp.float32),
        scratch_types=[
            pltpu.VMEM((_BPW,), jnp.int32),
            pltpu.VMEM((_NBUF, _CHUNK, EMBED_DIM), jnp.float32),
            pltpu.SemaphoreType.DMA,
            pltpu.SemaphoreType.DMA,
        ],
    )
    def gather_kernel(idx_hbm, table_hbm, out_hbm, idx_v, rows_v, gsem, osem):
        wid = lax.axis_index("s") * _NC + lax.axis_index("c")
        base = wid * _BPW
        pltpu.sync_copy(idx_hbm.at[pl.ds(base, _BPW)], idx_v)

        gathers = [None] * _NCHUNK
        stores = [None] * _NCHUNK
        for i in range(_NCHUNK):
            b = i % _NBUF
            if i >= _NBUF:
                stores[i - _NBUF].wait()
            gathers[i] = pltpu.async_copy(
                table_hbm.at[idx_v.at[pl.ds(i * _CHUNK, _CHUNK)]],
                rows_v.at[b],
                gsem,
            )
            if i > 0:
                gathers[i - 1].wait()
                stores[i - 1] = pltpu.async_copy(
                    rows_v.at[(i - 1) % _NBUF],
                    out_hbm.at[pl.ds(base + (i - 1) * _CHUNK, _CHUNK)],
                    osem,
                )
        gathers[_NCHUNK - 1].wait()
        stores[_NCHUNK - 1] = pltpu.async_copy(
            rows_v.at[(_NCHUNK - 1) % _NBUF],
            out_hbm.at[pl.ds(base + (_NCHUNK - 1) * _CHUNK, _CHUNK)],
            osem,
        )
        for i in range(max(0, _NCHUNK - _NBUF), _NCHUNK):
            stores[i].wait()

    return gather_kernel


_gather = _make_gather()


@jax.jit
def kernel(input, table):
    idx = jnp.reshape(input, (_B,)).astype(jnp.int32)
    out = _gather(idx, table)
    return jnp.reshape(out, (BATCH, SEQ, EMBED_DIM))
